# baseline (speedup 1.0000x reference)
import functools

import jax
import jax.numpy as jnp
import numpy as np
from jax import lax
from jax.experimental import pallas as pl
from jax.experimental.pallas import tpu as pltpu
from jax.experimental.pallas import tpu_sc as plsc

_FIELD_DIMS = [38462] * 26
_NUM_FIELDS = 26
_EMBED_DIM = 16
_BATCH = 16384
_VOCAB = 1000012

_NW = 32
_CW = 768
_NFULL = _VOCAB // _CW
_TAILC = _VOCAB - _NFULL * _CW
_TAILP = _TAILC + (-_TAILC % 8)
_K1BASE = _NFULL // _NW
_K1EXTRA = _NFULL - _K1BASE * _NW
_K1PAIRS = (_K1BASE + 2) // 2

_NCH = 128
_CPW = _NCH // _NW
_CHROWS = 128 * _NUM_FIELDS

_OFFSETS = np.concatenate(([0], np.cumsum(_FIELD_DIMS)[:-1])).astype(np.int32)
_OFF_CHUNK = _OFFSETS[np.arange(_CHROWS) % _NUM_FIELDS].reshape(_NUM_FIELDS, 128)
_M = np.arange(_NUM_FIELDS * _EMBED_DIM)
_ROWBASE = ((_M // 16) * 2048 + ((_M % 16) // 8) * 1024 + (_M % 8)).reshape(13, 32)
_ROWBASE = _ROWBASE.astype(np.int32)


def _tr_body(tab_hbm, tail_hbm, out_hbm,
             in_a, in_b, out_a, out_b, tailb,
             sia, sib, soa, sob, sem):
    wid = lax.axis_index("s") * 2 + lax.axis_index("c")
    iota16 = lax.iota(jnp.int32, 16)
    sidx0 = iota16 * 16
    zero16 = jnp.zeros((16,), jnp.int32)
    base = wid * _K1BASE + jnp.minimum(wid, _K1EXTRA)
    n = jnp.where(wid < _K1EXTRA, _K1BASE + 1, _K1BASE)

    def start_in(ck, buf, s):
        return pltpu.async_copy(tab_hbm.at[:, pl.ds(ck * _CW, _CW)], buf, s)

    def compute(inb, outb):
        def step(s, carry):
            b0 = zero16 + s * 256
            for d in range(16):
                vals = inb[d, pl.ds(s * 16, 16)]
                plsc.store_scatter(outb, [sidx0 + (b0 + d)], vals)
            return carry

        lax.fori_loop(0, _CW // 16, step, 0)

    def start_out(ck, buf, s):
        return pltpu.async_copy(
            buf, out_hbm.at[pl.ds(ck * (_CW * 16), _CW * 16)], s)

    def phase(i_rel, inb, outb, nbuf, si, snx, so, need_drain):
        ck = base + i_rel

        @pl.when(i_rel < n)
        def _():
            pltpu.make_async_copy(
                tab_hbm.at[:, pl.ds(0, _CW)], inb, si).wait()

            @pl.when(i_rel + 1 < n)
            def _():
                start_in(ck + 1, nbuf, snx)

            if need_drain:
                pltpu.make_async_copy(
                    out_hbm.at[pl.ds(0, _CW * 16)], outb, so).wait()
            compute(inb, outb)
            start_out(ck, outb, so)

    start_in(base, in_a, sia)
    for jp in range(_K1PAIRS):
        phase(2 * jp, in_a, out_a, in_b, sia, sib, soa, jp > 0)
        phase(2 * jp + 1, in_b, out_b, in_a, sib, sia, sob, jp > 0)

    @pl.when(n >= 1)
    def _():
        pltpu.make_async_copy(out_hbm.at[pl.ds(0, _CW * 16)], out_a, soa).wait()

    @pl.when(n >= 2)
    def _():
        pltpu.make_async_copy(out_hbm.at[pl.ds(0, _CW * 16)], out_b, sob).wait()

    @pl.when(wid == _NW - 1)
    def _():
        pltpu.async_copy(tail_hbm, tailb, sem).wait()

        def tail_step(v, carry):
            vals = plsc.load_gather(tailb, [zero16 + v, iota16])
            out_a[pl.ds(v * 16, 16)] = vals
            return carry

        lax.fori_loop(0, _TAILC, tail_step, 0)
        pltpu.async_copy(
            out_a.at[pl.ds(0, _TAILC * 16)],
            out_hbm.at[pl.ds(_NFULL * _CW * 16, _TAILC * 16)], sem).wait()


def _transpose_table(table):
    tab_t = jnp.transpose(table)
    tail = lax.slice(table, (_NFULL * _CW, 0), (_VOCAB, _EMBED_DIM))
    tail = jnp.pad(tail, ((0, _TAILP - _TAILC), (0, 0)))
    run = functools.partial(
        pl.kernel,
        out_type=jax.ShapeDtypeStruct((_VOCAB * _EMBED_DIM,), jnp.float32),
        mesh=plsc.VectorSubcoreMesh(core_axis_name="c", subcore_axis_name="s"),
        compiler_params=pltpu.CompilerParams(needs_layout_passes=False),
        scratch_types=[
            pltpu.VMEM((_EMBED_DIM, _CW), jnp.float32),
            pltpu.VMEM((_EMBED_DIM, _CW), jnp.float32),
            pltpu.VMEM((_CW * 16,), jnp.float32),
            pltpu.VMEM((_CW * 16,), jnp.float32),
            pltpu.VMEM((_TAILP, _EMBED_DIM), jnp.float32),
            pltpu.SemaphoreType.DMA,
            pltpu.SemaphoreType.DMA,
            pltpu.SemaphoreType.DMA,
            pltpu.SemaphoreType.DMA,
            pltpu.SemaphoreType.DMA,
        ],
    )(_tr_body)
    return run(tab_t, tail)


def _gather_body(x_hbm, off_hbm, tab_hbm, out_hbm,
                 xv, ov, rv, ob, sem, semb):
    wid = lax.axis_index("s") * 2 + lax.axis_index("c")
    iota16 = lax.iota(jnp.int32, 16)
    zero16 = jnp.zeros((16,), jnp.int32)
    pltpu.sync_copy(off_hbm, ov)

    def chunk(ci, carry):
        c = wid * _CPW + ci
        pltpu.sync_copy(x_hbm.at[c], xv)
        for j in range(_NUM_FIELDS):
            for i in range(8):
                sl = pl.ds(i * 16, 16)
                xv[j, sl] = xv[j, sl] + ov[j, sl]
        cps = [
            pltpu.async_copy(tab_hbm.at[xv.at[j]], rv.at[j],
                             sem if j < 13 else semb)
            for j in range(_NUM_FIELDS)
        ]

        def gstep(grp, carry2):
            base_pv = iota16 * _NUM_FIELDS + grp * 416
            sl = pl.ds(grp * 16, 16)
            for f in range(_NUM_FIELDS):
                pv = base_pv + f
                j2 = lax.shift_right_logical(pv, 7)
                k2 = jnp.bitwise_and(pv, 127)
                for d in range(16):
                    vals = plsc.load_gather(rv, [j2, k2, zero16 + d])
                    ob[f * 16 + d, sl] = vals
            return carry2

        for cp in cps[:13]:
            cp.wait()
        lax.fori_loop(0, 4, gstep, 0)
        for cp in cps[13:]:
            cp.wait()
        lax.fori_loop(4, 8, gstep, 0)

        c8 = c * 8
        sps = [
            pltpu.async_copy(
                ob.at[pl.ds(f * 16 + 8 * g, 8)],
                out_hbm.at[pl.ds(f * 2048 + g * 1024 + c8, 8)], sem)
            for f in range(_NUM_FIELDS) for g in range(2)
        ]
        for sp in sps:
            sp.wait()
        return carry

    lax.fori_loop(0, _CPW, chunk, 0)


def _gather(x3, table_rm):
    off = jnp.asarray(_OFF_CHUNK)
    run = functools.partial(
        pl.kernel,
        out_type=jax.ShapeDtypeStruct((53248, 128), jnp.float32),
        mesh=plsc.VectorSubcoreMesh(core_axis_name="c", subcore_axis_name="s"),
        compiler_params=pltpu.CompilerParams(
            use_tc_tiling_on_sc=False, needs_layout_passes=False),
        scratch_types=[
            pltpu.VMEM((_NUM_FIELDS, 128), jnp.int32),
            pltpu.VMEM((_NUM_FIELDS, 128), jnp.int32),
            pltpu.VMEM((_NUM_FIELDS, 128, _EMBED_DIM), jnp.float32),
            pltpu.VMEM((416, 128), jnp.float32),
            pltpu.SemaphoreType.DMA,
            pltpu.SemaphoreType.DMA,
        ],
    )(_gather_body)
    return run(x3, off, table_rm)


def kernel(x, table):
    table_rm = _transpose_table(table).reshape(_VOCAB, _EMBED_DIM)
    x3 = x.reshape(_NCH, _NUM_FIELDS, 128)
    out2d = _gather(x3, table_rm)
    out5 = out2d.reshape(_NUM_FIELDS, 2, 128, 8, 128)
    return jnp.transpose(out5, (2, 4, 0, 1, 3)).reshape(
        _BATCH, _NUM_FIELDS, _EMBED_DIM)


# speedup vs baseline: 1.2673x; 1.2673x over previous
import functools

import jax
import jax.numpy as jnp
import numpy as np
from jax import lax
from jax.experimental import pallas as pl
from jax.experimental.pallas import tpu as pltpu
from jax.experimental.pallas import tpu_sc as plsc

_FIELD_DIMS = [38462] * 26
_NUM_FIELDS = 26
_EMBED_DIM = 16
_BATCH = 16384
_VOCAB = 1000012

_NW = 32
_CW = 768
_NFULL = _VOCAB // _CW
_TAILC = _VOCAB - _NFULL * _CW
_TAILP = _TAILC + (-_TAILC % 8)
_K1BASE = _NFULL // _NW
_K1EXTRA = _NFULL - _K1BASE * _NW
_K1PAIRS = (_K1BASE + 2) // 2

_NCH = 128
_CPW = _NCH // _NW
_CHROWS = 128 * _NUM_FIELDS

_OFFSETS = np.concatenate(([0], np.cumsum(_FIELD_DIMS)[:-1])).astype(np.int32)
_OFF_CHUNK = _OFFSETS[np.arange(_CHROWS) % _NUM_FIELDS].reshape(_NUM_FIELDS, 128)
_M = np.arange(_NUM_FIELDS * _EMBED_DIM)
_ROWBASE = ((_M // 16) * 2048 + ((_M % 16) // 8) * 1024 + (_M % 8)).reshape(13, 32)
_ROWBASE = _ROWBASE.astype(np.int32)


def _tr_body(tab_hbm, tail_hbm, out_hbm,
             in_a, in_b, out_a, out_b, tailb,
             sia, sib, soa, sob, sem):
    wid = lax.axis_index("s") * 2 + lax.axis_index("c")
    iota16 = lax.iota(jnp.int32, 16)
    sidx0 = iota16 * 16
    zero16 = jnp.zeros((16,), jnp.int32)
    base = wid * _K1BASE + jnp.minimum(wid, _K1EXTRA)
    n = jnp.where(wid < _K1EXTRA, _K1BASE + 1, _K1BASE)

    def start_in(ck, buf, s):
        return pltpu.async_copy(tab_hbm.at[:, pl.ds(ck * _CW, _CW)], buf, s)

    def compute(inb, outb):
        def step(s, carry):
            b0 = zero16 + s * 256
            for d in range(16):
                vals = inb[d, pl.ds(s * 16, 16)]
                plsc.store_scatter(outb, [sidx0 + (b0 + d)], vals)
            return carry

        lax.fori_loop(0, _CW // 16, step, 0)

    def start_out(ck, buf, s):
        return pltpu.async_copy(
            buf, out_hbm.at[pl.ds(ck * (_CW * 16), _CW * 16)], s)

    def phase(i_rel, inb, outb, nbuf, si, snx, so, need_drain):
        ck = base + i_rel

        @pl.when(i_rel < n)
        def _():
            pltpu.make_async_copy(
                tab_hbm.at[:, pl.ds(0, _CW)], inb, si).wait()

            @pl.when(i_rel + 1 < n)
            def _():
                start_in(ck + 1, nbuf, snx)

            if need_drain:
                pltpu.make_async_copy(
                    out_hbm.at[pl.ds(0, _CW * 16)], outb, so).wait()
            compute(inb, outb)
            start_out(ck, outb, so)

    start_in(base, in_a, sia)
    for jp in range(_K1PAIRS):
        phase(2 * jp, in_a, out_a, in_b, sia, sib, soa, jp > 0)
        phase(2 * jp + 1, in_b, out_b, in_a, sib, sia, sob, jp > 0)

    @pl.when(n >= 1)
    def _():
        pltpu.make_async_copy(out_hbm.at[pl.ds(0, _CW * 16)], out_a, soa).wait()

    @pl.when(n >= 2)
    def _():
        pltpu.make_async_copy(out_hbm.at[pl.ds(0, _CW * 16)], out_b, sob).wait()

    @pl.when(wid == _NW - 1)
    def _():
        pltpu.async_copy(tail_hbm, tailb, sem).wait()

        def tail_step(v, carry):
            vals = plsc.load_gather(tailb, [zero16 + v, iota16])
            out_a[pl.ds(v * 16, 16)] = vals
            return carry

        lax.fori_loop(0, _TAILC, tail_step, 0)
        pltpu.async_copy(
            out_a.at[pl.ds(0, _TAILC * 16)],
            out_hbm.at[pl.ds(_NFULL * _CW * 16, _TAILC * 16)], sem).wait()


def _transpose_table(table):
    tab_t = jnp.transpose(table)
    tail = lax.slice(table, (_NFULL * _CW, 0), (_VOCAB, _EMBED_DIM))
    tail = jnp.pad(tail, ((0, _TAILP - _TAILC), (0, 0)))
    run = functools.partial(
        pl.kernel,
        out_type=jax.ShapeDtypeStruct((_VOCAB * _EMBED_DIM,), jnp.float32),
        mesh=plsc.VectorSubcoreMesh(core_axis_name="c", subcore_axis_name="s"),
        compiler_params=pltpu.CompilerParams(needs_layout_passes=False),
        scratch_types=[
            pltpu.VMEM((_EMBED_DIM, _CW), jnp.float32),
            pltpu.VMEM((_EMBED_DIM, _CW), jnp.float32),
            pltpu.VMEM((_CW * 16,), jnp.float32),
            pltpu.VMEM((_CW * 16,), jnp.float32),
            pltpu.VMEM((_TAILP, _EMBED_DIM), jnp.float32),
            pltpu.SemaphoreType.DMA,
            pltpu.SemaphoreType.DMA,
            pltpu.SemaphoreType.DMA,
            pltpu.SemaphoreType.DMA,
            pltpu.SemaphoreType.DMA,
        ],
    )(_tr_body)
    return run(tab_t, tail)


def _gather_body(x_hbm, off_hbm, tab_hbm, out_hbm,
                 xv, ov, rv, ob, sem, semb):
    wid = lax.axis_index("s") * 2 + lax.axis_index("c")
    iota16 = lax.iota(jnp.int32, 16)
    zero16 = jnp.zeros((16,), jnp.int32)
    pltpu.sync_copy(off_hbm, ov)

    def chunk(ci, carry):
        c = wid * _CPW + ci
        pltpu.sync_copy(x_hbm.at[c], xv)
        for j in range(_NUM_FIELDS):
            for i in range(8):
                sl = pl.ds(i * 16, 16)
                xv[j, sl] = xv[j, sl] + ov[j, sl]
        _ABLATE_GATHER = False
        cps = [] if _ABLATE_GATHER else [
            pltpu.async_copy(tab_hbm.at[xv.at[j]], rv.at[j],
                             sem if j < 13 else semb)
            for j in range(_NUM_FIELDS)
        ]

        def gstep(grp, carry2):
            base_pv = iota16 * _NUM_FIELDS + grp * 416
            sl = pl.ds(grp * 16, 16)
            for f in range(_NUM_FIELDS):
                pv = base_pv + f
                j2 = lax.shift_right_logical(pv, 7)
                k2 = jnp.bitwise_and(pv, 127)
                vals = [
                    plsc.load_gather(rv, [j2, k2, zero16 + d])
                    for d in range(16)
                ]
                for d in range(16):
                    ob[f * 16 + d, sl] = vals[d]
            return carry2

        _ABLATE_TR = False
        for cp in cps[:13]:
            cp.wait()
        if not _ABLATE_TR:
            lax.fori_loop(0, 4, gstep, 0)
        for cp in cps[13:]:
            cp.wait()
        if not _ABLATE_TR:
            lax.fori_loop(4, 8, gstep, 0)

        c8 = c * 8
        sps = [
            pltpu.async_copy(
                ob.at[pl.ds(f * 16 + 8 * g, 8)],
                out_hbm.at[pl.ds(f * 2048 + g * 1024 + c8, 8)], sem)
            for f in range(_NUM_FIELDS) for g in range(2)
        ]
        for sp in sps:
            sp.wait()
        return carry

    lax.fori_loop(0, _CPW, chunk, 0)


def _gather(x3, table_rm):
    off = jnp.asarray(_OFF_CHUNK)
    run = functools.partial(
        pl.kernel,
        out_type=jax.ShapeDtypeStruct((53248, 128), jnp.float32),
        mesh=plsc.VectorSubcoreMesh(core_axis_name="c", subcore_axis_name="s"),
        compiler_params=pltpu.CompilerParams(
            use_tc_tiling_on_sc=False, needs_layout_passes=False),
        scratch_types=[
            pltpu.VMEM((_NUM_FIELDS, 128), jnp.int32),
            pltpu.VMEM((_NUM_FIELDS, 128), jnp.int32),
            pltpu.VMEM((_NUM_FIELDS, 128, _EMBED_DIM), jnp.float32),
            pltpu.VMEM((416, 128), jnp.float32),
            pltpu.SemaphoreType.DMA,
            pltpu.SemaphoreType.DMA,
        ],
    )(_gather_body)
    return run(x3, off, table_rm)


def kernel(x, table):
    table_rm = _transpose_table(table).reshape(_VOCAB, _EMBED_DIM)
    x3 = x.reshape(_NCH, _NUM_FIELDS, 128)
    out2d = _gather(x3, table_rm)
    out5 = out2d.reshape(_NUM_FIELDS, 2, 128, 8, 128)
    return jnp.transpose(out5, (2, 4, 0, 1, 3)).reshape(
        _BATCH, _NUM_FIELDS, _EMBED_DIM)


# speedup vs baseline: 1.5856x; 1.2512x over previous
import functools

import jax
import jax.numpy as jnp
import numpy as np
from jax import lax
from jax.experimental import pallas as pl
from jax.experimental.pallas import tpu as pltpu
from jax.experimental.pallas import tpu_sc as plsc

_FIELD_DIMS = [38462] * 26
_NUM_FIELDS = 26
_EMBED_DIM = 16
_BATCH = 16384
_VOCAB = 1000012

_NW = 32
_CW = 768
_NFULL = _VOCAB // _CW
_TAILC = _VOCAB - _NFULL * _CW
_TAILP = _TAILC + (-_TAILC % 8)
_K1BASE = _NFULL // _NW
_K1EXTRA = _NFULL - _K1BASE * _NW
_K1PAIRS = (_K1BASE + 2) // 2

_NCH = 128
_CPW = _NCH // _NW
_CHROWS = 128 * _NUM_FIELDS

_OFFSETS = np.concatenate(([0], np.cumsum(_FIELD_DIMS)[:-1])).astype(np.int32)
_OFF_CHUNK = _OFFSETS[np.arange(_CHROWS) % _NUM_FIELDS].reshape(_NUM_FIELDS, 128)
_M = np.arange(_NUM_FIELDS * _EMBED_DIM)
_ROWBASE = ((_M // 16) * 2048 + ((_M % 16) // 8) * 1024 + (_M % 8)).reshape(13, 32)
_ROWBASE = _ROWBASE.astype(np.int32)


def _tr_body(tab_hbm, tail_hbm, out_hbm,
             in_a, in_b, out_a, out_b, tailb,
             sia, sib, soa, sob, sem):
    wid = lax.axis_index("s") * 2 + lax.axis_index("c")
    iota16 = lax.iota(jnp.int32, 16)
    sidx0 = iota16 * 16
    zero16 = jnp.zeros((16,), jnp.int32)
    base = wid * _K1BASE + jnp.minimum(wid, _K1EXTRA)
    n = jnp.where(wid < _K1EXTRA, _K1BASE + 1, _K1BASE)

    def start_in(ck, buf, s):
        return pltpu.async_copy(tab_hbm.at[:, pl.ds(ck * _CW, _CW)], buf, s)

    def compute(inb, outb):
        def step(s, carry):
            b0 = zero16 + s * 256
            vals = [inb[d, pl.ds(s * 16, 16)] for d in range(16)]
            for d in range(16):
                plsc.store_scatter(outb, [sidx0 + (b0 + d)], vals[d])
            return carry

        lax.fori_loop(0, _CW // 16, step, 0)

    def start_out(ck, buf, s):
        return pltpu.async_copy(
            buf, out_hbm.at[pl.ds(ck * (_CW * 16), _CW * 16)], s)

    def phase(i_rel, inb, outb, nbuf, si, snx, so, need_drain):
        ck = base + i_rel

        @pl.when(i_rel < n)
        def _():
            pltpu.make_async_copy(
                tab_hbm.at[:, pl.ds(0, _CW)], inb, si).wait()

            @pl.when(i_rel + 1 < n)
            def _():
                start_in(ck + 1, nbuf, snx)

            if need_drain:
                pltpu.make_async_copy(
                    out_hbm.at[pl.ds(0, _CW * 16)], outb, so).wait()
            compute(inb, outb)
            start_out(ck, outb, so)

    start_in(base, in_a, sia)
    for jp in range(_K1PAIRS):
        phase(2 * jp, in_a, out_a, in_b, sia, sib, soa, jp > 0)
        phase(2 * jp + 1, in_b, out_b, in_a, sib, sia, sob, jp > 0)

    @pl.when(n >= 1)
    def _():
        pltpu.make_async_copy(out_hbm.at[pl.ds(0, _CW * 16)], out_a, soa).wait()

    @pl.when(n >= 2)
    def _():
        pltpu.make_async_copy(out_hbm.at[pl.ds(0, _CW * 16)], out_b, sob).wait()

    @pl.when(wid == _NW - 1)
    def _():
        pltpu.async_copy(tail_hbm, tailb, sem).wait()

        def tail_step(v, carry):
            vals = plsc.load_gather(tailb, [zero16 + v, iota16])
            out_a[pl.ds(v * 16, 16)] = vals
            return carry

        lax.fori_loop(0, _TAILC, tail_step, 0)
        pltpu.async_copy(
            out_a.at[pl.ds(0, _TAILC * 16)],
            out_hbm.at[pl.ds(_NFULL * _CW * 16, _TAILC * 16)], sem).wait()


def _transpose_table(table):
    tab_t = jnp.transpose(table)
    tail = lax.slice(table, (_NFULL * _CW, 0), (_VOCAB, _EMBED_DIM))
    tail = jnp.pad(tail, ((0, _TAILP - _TAILC), (0, 0)))
    run = functools.partial(
        pl.kernel,
        out_type=jax.ShapeDtypeStruct((_VOCAB * _EMBED_DIM,), jnp.float32),
        mesh=plsc.VectorSubcoreMesh(core_axis_name="c", subcore_axis_name="s"),
        compiler_params=pltpu.CompilerParams(needs_layout_passes=False),
        scratch_types=[
            pltpu.VMEM((_EMBED_DIM, _CW), jnp.float32),
            pltpu.VMEM((_EMBED_DIM, _CW), jnp.float32),
            pltpu.VMEM((_CW * 16,), jnp.float32),
            pltpu.VMEM((_CW * 16,), jnp.float32),
            pltpu.VMEM((_TAILP, _EMBED_DIM), jnp.float32),
            pltpu.SemaphoreType.DMA,
            pltpu.SemaphoreType.DMA,
            pltpu.SemaphoreType.DMA,
            pltpu.SemaphoreType.DMA,
            pltpu.SemaphoreType.DMA,
        ],
    )(_tr_body)
    return run(tab_t, tail)


def _gather_body(x_hbm, off_hbm, tab_hbm, out_hbm,
                 xv, ov, rv, ob, sem, semb):
    wid = lax.axis_index("s") * 2 + lax.axis_index("c")
    iota16 = lax.iota(jnp.int32, 16)
    zero16 = jnp.zeros((16,), jnp.int32)
    pltpu.sync_copy(off_hbm, ov)

    def chunk(ci, carry):
        c = wid * _CPW + ci
        pltpu.sync_copy(x_hbm.at[c], xv)
        for j in range(_NUM_FIELDS):
            for i in range(8):
                sl = pl.ds(i * 16, 16)
                xv[j, sl] = xv[j, sl] + ov[j, sl]
        _ABLATE_GATHER = False
        cps = [] if _ABLATE_GATHER else [
            pltpu.async_copy(tab_hbm.at[xv.at[j]], rv.at[j],
                             sem if j < 13 else semb)
            for j in range(_NUM_FIELDS)
        ]

        def gstep(grp, carry2):
            base_pv = iota16 * _NUM_FIELDS + grp * 416
            sl = pl.ds(grp * 16, 16)
            for f0 in range(0, _NUM_FIELDS, 2):
                vals = []
                for f in (f0, f0 + 1):
                    pv = base_pv + f
                    j2 = lax.shift_right_logical(pv, 7)
                    k2 = jnp.bitwise_and(pv, 127)
                    vals += [
                        plsc.load_gather(rv, [j2, k2, zero16 + d])
                        for d in range(16)
                    ]
                for i, v in enumerate(vals):
                    ob[f0 * 16 + i, sl] = v
            return carry2

        _ABLATE_TR = False
        for cp in cps[:13]:
            cp.wait()
        if not _ABLATE_TR:
            lax.fori_loop(0, 4, gstep, 0)
        for cp in cps[13:]:
            cp.wait()
        if not _ABLATE_TR:
            lax.fori_loop(4, 8, gstep, 0)

        c8 = c * 8
        sps = [
            pltpu.async_copy(
                ob.at[pl.ds(f * 16 + 8 * g, 8)],
                out_hbm.at[pl.ds(f * 2048 + g * 1024 + c8, 8)], sem)
            for f in range(_NUM_FIELDS) for g in range(2)
        ]
        for sp in sps:
            sp.wait()
        return carry

    lax.fori_loop(0, _CPW, chunk, 0)


def _gather(x3, table_rm):
    off = jnp.asarray(_OFF_CHUNK)
    run = functools.partial(
        pl.kernel,
        out_type=jax.ShapeDtypeStruct((53248, 128), jnp.float32),
        mesh=plsc.VectorSubcoreMesh(core_axis_name="c", subcore_axis_name="s"),
        compiler_params=pltpu.CompilerParams(
            use_tc_tiling_on_sc=False, needs_layout_passes=False),
        scratch_types=[
            pltpu.VMEM((_NUM_FIELDS, 128), jnp.int32),
            pltpu.VMEM((_NUM_FIELDS, 128), jnp.int32),
            pltpu.VMEM((_NUM_FIELDS, 128, _EMBED_DIM), jnp.float32),
            pltpu.VMEM((416, 128), jnp.float32),
            pltpu.SemaphoreType.DMA,
            pltpu.SemaphoreType.DMA,
        ],
    )(_gather_body)
    return run(x3, off, table_rm)


def kernel(x, table):
    table_rm = _transpose_table(table).reshape(_VOCAB, _EMBED_DIM)
    x3 = x.reshape(_NCH, _NUM_FIELDS, 128)
    out2d = _gather(x3, table_rm)
    out5 = out2d.reshape(_NUM_FIELDS, 2, 128, 8, 128)
    return jnp.transpose(out5, (2, 4, 0, 1, 3)).reshape(
        _BATCH, _NUM_FIELDS, _EMBED_DIM)
